# Initial kernel scaffold; baseline (speedup 1.0000x reference)
import jax
import jax.numpy as jnp
from jax.experimental import pallas as pl


def kernel(x, token_table, pos_table):
    raise NotImplementedError("write your pallas kernel here")


# speedup vs baseline: 3.8325x; 3.8325x over previous
import functools

import jax
import jax.numpy as jnp
from jax import lax
from jax.experimental import pallas as pl
from jax.experimental.pallas import tpu as pltpu
from jax.experimental.pallas import tpu_sc as plsc

MAXLEN = 200
EMBED = 128
BATCH = 4096

_info = plsc.get_sparse_core_info()
NC, NS, L = _info.num_cores, _info.num_subcores, _info.num_lanes
NW = NC * NS
SEQ_PER_W = BATCH // NW
VPR = EMBED // L
IDX_CHUNKS = 2
IDX_MINOR = MAXLEN // IDX_CHUNKS


def _body(x_hbm, tok_hbm, pos_hbm, out_hbm, idx_v, buf, pos_v, sem):
    wid = lax.axis_index("s") * NC + lax.axis_index("c")

    pltpu.sync_copy(pos_hbm, pos_v)

    def one_seq(s, carry):
        seq = wid * SEQ_PER_W + s
        pltpu.sync_copy(x_hbm.at[seq], idx_v)
        for j in range(IDX_CHUNKS):
            pltpu.async_copy(tok_hbm.at[idx_v.at[j]],
                             buf.at[pl.ds(j * IDX_MINOR, IDX_MINOR)], sem).wait()

        def add_row(r, c2):
            for c in range(VPR):
                sl = pl.ds(c * L, L)
                buf[r, sl] = buf[r, sl] + pos_v[r, sl]
            return c2
        lax.fori_loop(0, MAXLEN, add_row, 0, unroll=False)

        pltpu.sync_copy(buf, out_hbm.at[pl.ds(seq * MAXLEN, MAXLEN)])
        return carry

    lax.fori_loop(0, SEQ_PER_W, one_seq, 0, unroll=False)


@functools.partial(jax.jit, static_argnames=())
def kernel(x, token_table, pos_table):
    x3 = x.astype(jnp.int32).reshape(BATCH, IDX_CHUNKS, IDX_MINOR)
    mesh = plsc.VectorSubcoreMesh(core_axis_name="c", subcore_axis_name="s")
    run = pl.kernel(
        _body,
        mesh=mesh,
        out_type=jax.ShapeDtypeStruct((BATCH * MAXLEN, EMBED), jnp.float32),
        scratch_types=[
            pltpu.VMEM((IDX_CHUNKS, IDX_MINOR), jnp.int32),
            pltpu.VMEM((MAXLEN, EMBED), jnp.float32),
            pltpu.VMEM((MAXLEN, EMBED), jnp.float32),
            pltpu.SemaphoreType.DMA,
        ],
    )
    out = run(x3, token_table, pos_table)
    return out.reshape(BATCH, MAXLEN, EMBED)


# speedup vs baseline: 9.1393x; 2.3847x over previous
import functools

import jax
import jax.numpy as jnp
from jax import lax
from jax.experimental import pallas as pl
from jax.experimental.pallas import tpu as pltpu
from jax.experimental.pallas import tpu_sc as plsc

MAXLEN = 200
EMBED = 128
BATCH = 4096

_info = plsc.get_sparse_core_info()
NC, NS, L = _info.num_cores, _info.num_subcores, _info.num_lanes
NW = NC * NS
SEQ_PER_W = BATCH // NW
VPR = EMBED // L
IDX_CHUNKS = 2
IDX_MINOR = MAXLEN // IDX_CHUNKS
NSLOT = 3


def _body(x_hbm, tok_hbm, pos_hbm, out_hbm,
          idx_ring, buf, pos_v, g0, g1, g2, o0, o1, o2, i0, i1, i2):
    gsem = (g0, g1, g2)
    osem = (o0, o1, o2)
    isem = (i0, i1, i2)
    wid = lax.axis_index("s") * NC + lax.axis_index("c")
    out_base = wid * SEQ_PER_W * MAXLEN

    pltpu.sync_copy(pos_hbm, pos_v)

    def fetch_idx(s, slot):
        pltpu.async_copy(x_hbm.at[wid, s], idx_ring.at[slot], isem[slot])

    def wait_idx(slot):
        pltpu.make_async_copy(x_hbm.at[wid, 0], idx_ring.at[slot],
                              isem[slot]).wait()

    def start_gather(slot):
        for j in range(IDX_CHUNKS):
            pltpu.async_copy(tok_hbm.at[idx_ring.at[slot, j]],
                             buf.at[slot, pl.ds(j * IDX_MINOR, IDX_MINOR)],
                             gsem[slot])

    def wait_gather(slot):
        for j in range(IDX_CHUNKS):
            pltpu.make_async_copy(tok_hbm.at[idx_ring.at[0, j]],
                                  buf.at[slot, pl.ds(j * IDX_MINOR, IDX_MINOR)],
                                  gsem[slot]).wait()

    def add_pos(slot):
        def add_row(r, carry):
            for cc in range(VPR):
                sl = pl.ds(cc * L, L)
                plsc.addupdate(buf.at[slot, r, sl], pos_v[r, sl])
            return carry
        lax.fori_loop(0, MAXLEN, add_row, 0, unroll=False)

    def start_out(s, slot):
        pltpu.async_copy(buf.at[slot],
                         out_hbm.at[pl.ds(out_base + s * MAXLEN, MAXLEN)],
                         osem[slot])

    def wait_out(slot):
        pltpu.make_async_copy(buf.at[slot],
                              out_hbm.at[pl.ds(0, MAXLEN)],
                              osem[slot]).wait()

    for s in range(NSLOT):
        fetch_idx(s, s)
    for s in range(NSLOT - 1):
        wait_idx(s)
        start_gather(s)

    def step(k, carry):
        for r in range(NSLOT):
            s = NSLOT * k + r

            @pl.when(s < SEQ_PER_W)
            def _compute():
                wait_gather(r)
                add_pos(r)
                start_out(s, r)

            @pl.when(s + NSLOT < SEQ_PER_W)
            def _refill():
                fetch_idx(s + NSLOT, r)

            pslot = (r + NSLOT - 1) % NSLOT
            pf_ok = s + NSLOT - 1 < SEQ_PER_W
            if r == 0:
                @pl.when(pf_ok & (k > 0))
                def _drain0():
                    wait_out(pslot)
            else:
                @pl.when(pf_ok)
                def _drain():
                    wait_out(pslot)

            @pl.when(pf_ok)
            def _prefetch():
                wait_idx(pslot)
                start_gather(pslot)
        return carry

    lax.fori_loop(0, (SEQ_PER_W + NSLOT) // NSLOT, step, 0, unroll=False)

    for s in range(SEQ_PER_W - NSLOT, SEQ_PER_W):
        wait_out(s % NSLOT)


@functools.partial(jax.jit, static_argnames=())
def kernel(x, token_table, pos_table):
    x4 = x.astype(jnp.int32).reshape(NW, SEQ_PER_W, IDX_CHUNKS, IDX_MINOR)
    mesh = plsc.VectorSubcoreMesh(core_axis_name="c", subcore_axis_name="s")
    run = pl.kernel(
        _body,
        mesh=mesh,
        out_type=jax.ShapeDtypeStruct((BATCH * MAXLEN, EMBED), jnp.float32),
        scratch_types=[
            pltpu.VMEM((NSLOT, IDX_CHUNKS, IDX_MINOR), jnp.int32),
            pltpu.VMEM((NSLOT, MAXLEN, EMBED), jnp.float32),
            pltpu.VMEM((MAXLEN, EMBED), jnp.float32),
            pltpu.SemaphoreType.DMA,
            pltpu.SemaphoreType.DMA,
            pltpu.SemaphoreType.DMA,
            pltpu.SemaphoreType.DMA,
            pltpu.SemaphoreType.DMA,
            pltpu.SemaphoreType.DMA,
            pltpu.SemaphoreType.DMA,
            pltpu.SemaphoreType.DMA,
            pltpu.SemaphoreType.DMA,
        ],
    )
    out = run(x4, token_table, pos_table)
    return out.reshape(BATCH, MAXLEN, EMBED)
